# baseline (speedup 1.0000x reference)
import functools

import jax
import jax.numpy as jnp
from jax import lax
from jax.experimental import pallas as pl
from jax.experimental.pallas import tpu as pltpu
from jax.experimental.pallas import tpu_sc as plsc

N = 10000
E = 320000
H = 128
B = 64

_NC = 2
_NS = 16
_NT = _NC * _NS
_CH = 80
_ITERS = 128
_D = 4
_CPT = _ITERS + 2
_EPT = _ITERS * _CH
_EPAD = _NT * _EPT
_NPAD = 10240
_DUMMY = N + 100
_RPT = _NPAD // _NS


def _segsum_sc(h, src, dst, zeros):
    mesh = plsc.VectorSubcoreMesh(core_axis_name="c", subcore_axis_name="s",
                                  num_cores=_NC, num_subcores=_NS)

    @functools.partial(
        pl.kernel,
        mesh=mesh,
        out_type=jax.ShapeDtypeStruct((_NC * _NPAD, H), jnp.float32),
        scratch_types=[
            [pltpu.VMEM((_CH,), jnp.int32) for _ in range(_D)],
            [pltpu.VMEM((_CH,), jnp.int32) for _ in range(_D)],
            pltpu.VMEM((_D, _CH, H), jnp.float32),
            pltpu.VMEM_SHARED((_NPAD, H), jnp.float32),
            [pltpu.SemaphoreType.DMA for _ in range(_D)],
            [pltpu.SemaphoreType.DMA for _ in range(_D)],
        ],
    )
    def k(h_hbm, src_hbm, dst_hbm, z_hbm, out_hbm, sidx, didx, rows, acc,
          isem, gsem):
        c = lax.axis_index("c")
        s = lax.axis_index("s")
        w = c * _NS + s
        pltpu.sync_copy(z_hbm.at[pl.ds(s * _RPT, _RPT)],
                        acc.at[pl.ds(s * _RPT, _RPT)])
        base = w * _CPT * _CH

        def idx_pair(j, b):
            return (pltpu.make_async_copy(
                        src_hbm.at[pl.ds(base + j * _CH, _CH)], sidx[b], isem[b]),
                    pltpu.make_async_copy(
                        dst_hbm.at[pl.ds(base + j * _CH, _CH)], didx[b], isem[b]))

        def gath(b):
            return pltpu.make_async_copy(h_hbm.at[sidx[b]], rows.at[b], gsem[b])

        def scat(b):
            gath(b).wait()
            pltpu.sync_copy(rows.at[b], acc.at[didx[b]], add=True)

        for d in idx_pair(0, 0) + idx_pair(1, 1):
            d.start()
        plsc.subcore_barrier()

        def body(j, b, do_scatter):
            if do_scatter:
                scat((b + 2) % _D)
            for d in idx_pair(j + 2, (b + 2) % _D):
                d.start()
            for d in idx_pair(0, b):
                d.wait()
            gath(b).start()

        body(0, 0, False)
        body(1, 1, False)
        body(2, 2, True)
        body(3, 3, True)

        def group(g, carry):
            for b in range(_D):
                body(g * _D + b, b, True)
            return carry

        lax.fori_loop(1, _ITERS // _D, group, 0)
        scat(2)
        scat(3)
        for b in (0, 1):
            for d in idx_pair(0, b):
                d.wait()
        plsc.subcore_barrier()
        pltpu.sync_copy(acc.at[pl.ds(s * _RPT, _RPT)],
                        out_hbm.at[pl.ds(c * _NPAD + s * _RPT, _RPT)])

    return k(h, src, dst, zeros)


_BLK = 1000


def _dense_pool(p, hprev, Wrel, brel_r, Wroot, batch_col):
    grid = (N // _BLK,)

    def body(p_ref, hp_ref, wr_ref, br_ref, wq_ref, b_ref,
             h_ref, mx_ref, sm_ref, cnt_ref):
        i = pl.program_id(0)
        agg = p_ref[0] + p_ref[1]
        hnew = lax.dot_general(agg, wr_ref[...], (((1,), (1,)), ((), ())),
                               preferred_element_type=jnp.float32)
        hnew = hnew + br_ref[...]
        hnew = hnew + lax.dot_general(hp_ref[...], wq_ref[...],
                                      (((1,), (1,)), ((), ())),
                                      preferred_element_type=jnp.float32)
        hnew = jnp.maximum(hnew, 0.0)
        h_ref[...] = hnew

        @pl.when(i == 0)
        def _init():
            mx_ref[...] = jnp.full((B, H), -jnp.inf, jnp.float32)
            sm_ref[...] = jnp.zeros((B, H), jnp.float32)
            cnt_ref[...] = jnp.zeros((B, H), jnp.float32)

        bb = b_ref[...]
        s_lo = b_ref[0, 0].astype(jnp.int32)
        s_hi = b_ref[_BLK - 1, 0].astype(jnp.int32)

        def seg(sgi, carry):
            m = bb == sgi.astype(jnp.float32)
            hmask = jnp.where(m, hnew, -jnp.inf)
            hzero = jnp.where(m, hnew, 0.0)
            mx_ref[pl.ds(sgi, 1), :] = jnp.maximum(
                mx_ref[pl.ds(sgi, 1), :], jnp.max(hmask, axis=0, keepdims=True))
            sm_ref[pl.ds(sgi, 1), :] = (
                sm_ref[pl.ds(sgi, 1), :] + jnp.sum(hzero, axis=0, keepdims=True))
            cnt_ref[pl.ds(sgi, 1), :] = (
                cnt_ref[pl.ds(sgi, 1), :] + jnp.sum(m.astype(jnp.float32)))
            return carry

        lax.fori_loop(s_lo, s_hi + 1, seg, 0)

    return pl.pallas_call(
        body,
        grid=grid,
        in_specs=[
            pl.BlockSpec((2, _BLK, H), lambda i: (0, i, 0)),
            pl.BlockSpec((_BLK, H), lambda i: (i, 0)),
            pl.BlockSpec((H, H), lambda i: (0, 0)),
            pl.BlockSpec((1, H), lambda i: (0, 0)),
            pl.BlockSpec((H, H), lambda i: (0, 0)),
            pl.BlockSpec((_BLK, 1), lambda i: (i, 0)),
        ],
        out_specs=[
            pl.BlockSpec((_BLK, H), lambda i: (i, 0)),
            pl.BlockSpec((B, H), lambda i: (0, 0)),
            pl.BlockSpec((B, H), lambda i: (0, 0)),
            pl.BlockSpec((B, H), lambda i: (0, 0)),
        ],
        out_shape=[
            jax.ShapeDtypeStruct((N, H), jnp.float32),
            jax.ShapeDtypeStruct((B, H), jnp.float32),
            jax.ShapeDtypeStruct((B, H), jnp.float32),
            jax.ShapeDtypeStruct((B, H), jnp.float32),
        ],
    )(p.reshape(_NC, _NPAD, H), hprev, Wrel, brel_r, Wroot, batch_col)


def _mlp(mx1, sm1, cnt1, mx2, sm2, cnt2, mx3, sm3, cnt3,
         W1, b1_r, W2, b2_r, W3, b3_r):
    def body(mx1_ref, sm1_ref, cnt1_ref, mx2_ref, sm2_ref, cnt2_ref,
             mx3_ref, sm3_ref, cnt3_ref, w1_ref, b1_ref, w2_ref, b2_ref,
             w3_ref, b3_ref, out_ref, enc_ref):
        def gpart(mx_ref, sm_ref, cnt_ref):
            cnt = jnp.maximum(cnt_ref[...], 1.0)
            return jnp.concatenate([mx_ref[...], sm_ref[...] / cnt], axis=1)

        g = (gpart(mx1_ref, sm1_ref, cnt1_ref)
             + gpart(mx2_ref, sm2_ref, cnt2_ref)
             + gpart(mx3_ref, sm3_ref, cnt3_ref))
        enc_ref[...] = g
        z = lax.dot_general(g, w1_ref[...], (((1,), (1,)), ((), ())),
                            preferred_element_type=jnp.float32) + b1_ref[...]
        z = jnp.maximum(z, 0.0)
        z = lax.dot_general(z, w2_ref[...], (((1,), (1,)), ((), ())),
                            preferred_element_type=jnp.float32) + b2_ref[...]
        z = jnp.maximum(z, 0.0)
        z = jnp.sum(z * w3_ref[...], axis=1, keepdims=True) + b3_ref[0, 0]
        out_ref[...] = z

    return pl.pallas_call(
        body,
        out_shape=[
            jax.ShapeDtypeStruct((B, 1), jnp.float32),
            jax.ShapeDtypeStruct((B, 2 * H), jnp.float32),
        ],
    )(mx1, sm1, cnt1, mx2, sm2, cnt2, mx3, sm3, cnt3,
      W1, b1_r, W2, b2_r, W3, b3_r)


def kernel(x, edge_index, batch, Wrel1, brel1, Wroot1, Wrel2, brel2, Wroot2,
           Wrel3, brel3, Wroot3, W_lin1, b_lin1, W_lin2, b_lin2, W_lin3, b_lin3):
    pad = _EPAD - E
    dpad = (_CPT - _ITERS) * _CH
    src = jnp.concatenate([edge_index[0], jnp.zeros((pad,), jnp.int32)])
    src = jnp.concatenate(
        [src.reshape(_NT, _EPT), jnp.zeros((_NT, dpad), jnp.int32)], axis=1
    ).reshape(-1)
    dst = jnp.concatenate([edge_index[1], jnp.full((pad,), _DUMMY, jnp.int32)])
    dst = jnp.concatenate(
        [dst.reshape(_NT, _EPT), jnp.full((_NT, dpad), _DUMMY, jnp.int32)],
        axis=1).reshape(-1)
    zeros = jnp.zeros((_NPAD, H), jnp.float32)
    batch_col = batch.astype(jnp.float32).reshape(N, 1)

    p1 = _segsum_sc(x, src, dst, zeros)
    h1, mx1, sm1, cnt1 = _dense_pool(p1, x, Wrel1, brel1.reshape(1, H),
                                     Wroot1, batch_col)
    p2 = _segsum_sc(h1, src, dst, zeros)
    h2, mx2, sm2, cnt2 = _dense_pool(p2, h1, Wrel2, brel2.reshape(1, H),
                                     Wroot2, batch_col)
    p3 = _segsum_sc(h2, src, dst, zeros)
    h3, mx3, sm3, cnt3 = _dense_pool(p3, h2, Wrel3, brel3.reshape(1, H),
                                     Wroot3, batch_col)

    out, enc = _mlp(mx1, sm1, cnt1, mx2, sm2, cnt2, mx3, sm3, cnt3,
                    W_lin1, b_lin1.reshape(1, -1),
                    W_lin2, b_lin2.reshape(1, -1),
                    W_lin3, b_lin3.reshape(1, -1))
    return (out, lax.stop_gradient(enc))


# speedup vs baseline: 1.3614x; 1.3614x over previous
import functools

import jax
import jax.numpy as jnp
from jax import lax
from jax.experimental import pallas as pl
from jax.experimental.pallas import tpu as pltpu
from jax.experimental.pallas import tpu_sc as plsc

N = 10000
E = 320000
H = 128
B = 64

_NC = 2
_NS = 16
_NT = _NC * _NS
_CH = 104
_ITERS = 97
_CPT = _ITERS
_EPT = _ITERS * _CH
_EPAD = _NT * _EPT
_NPAD = 10240
_DUMMY = N + 100
_RPT = _NPAD // _NS


def _segsum_sc(h, src, dst, zeros):
    mesh = plsc.VectorSubcoreMesh(core_axis_name="c", subcore_axis_name="s",
                                  num_cores=_NC, num_subcores=_NS)

    @functools.partial(
        pl.kernel,
        mesh=mesh,
        out_type=jax.ShapeDtypeStruct((_NC * _NPAD, H), jnp.float32),
        scratch_types=[
            pltpu.VMEM((_CH,), jnp.int32),
            pltpu.VMEM((_CH,), jnp.int32),
            pltpu.VMEM((_CH, H), jnp.float32),
            pltpu.VMEM_SHARED((_NPAD, H), jnp.float32),
            pltpu.SemaphoreType.DMA,
            pltpu.SemaphoreType.DMA,
        ],
    )
    def k(h_hbm, src_hbm, dst_hbm, z_hbm, out_hbm, sidx, didx, rows, acc,
          isem, gsem):
        c = lax.axis_index("c")
        s = lax.axis_index("s")
        w = c * _NS + s
        pltpu.sync_copy(z_hbm.at[pl.ds(s * _RPT, _RPT)],
                        acc.at[pl.ds(s * _RPT, _RPT)])
        base = w * _CPT * _CH
        plsc.subcore_barrier()

        def idx_pair(j):
            return (pltpu.make_async_copy(
                        src_hbm.at[pl.ds(base + j * _CH, _CH)], sidx, isem),
                    pltpu.make_async_copy(
                        dst_hbm.at[pl.ds(base + j * _CH, _CH)], didx, isem))

        def body(j, carry):
            for d in idx_pair(j):
                d.start()
            for d in idx_pair(j):
                d.wait()
            pltpu.async_copy(h_hbm.at[sidx], rows, gsem).wait()
            pltpu.sync_copy(rows, acc.at[didx], add=True)
            return carry

        lax.fori_loop(0, _ITERS, body, 0)
        plsc.subcore_barrier()
        pltpu.sync_copy(acc.at[pl.ds(s * _RPT, _RPT)],
                        out_hbm.at[pl.ds(c * _NPAD + s * _RPT, _RPT)])

    return k(h, src, dst, zeros)


_BLK = 1000


def _dense_pool(p, hprev, Wrel, brel_r, Wroot, batch_col):
    grid = (N // _BLK,)

    def body(p_ref, hp_ref, wr_ref, br_ref, wq_ref, b_ref,
             h_ref, mx_ref, sm_ref, cnt_ref):
        i = pl.program_id(0)
        agg = p_ref[0] + p_ref[1]
        hnew = lax.dot_general(agg, wr_ref[...], (((1,), (1,)), ((), ())),
                               preferred_element_type=jnp.float32)
        hnew = hnew + br_ref[...]
        hnew = hnew + lax.dot_general(hp_ref[...], wq_ref[...],
                                      (((1,), (1,)), ((), ())),
                                      preferred_element_type=jnp.float32)
        hnew = jnp.maximum(hnew, 0.0)
        h_ref[...] = hnew

        @pl.when(i == 0)
        def _init():
            mx_ref[...] = jnp.full((B, H), -jnp.inf, jnp.float32)
            sm_ref[...] = jnp.zeros((B, H), jnp.float32)
            cnt_ref[...] = jnp.zeros((B, H), jnp.float32)

        bb = b_ref[...]
        s_lo = b_ref[0, 0].astype(jnp.int32)
        s_hi = b_ref[_BLK - 1, 0].astype(jnp.int32)

        def seg(sgi, carry):
            m = bb == sgi.astype(jnp.float32)
            hmask = jnp.where(m, hnew, -jnp.inf)
            hzero = jnp.where(m, hnew, 0.0)
            mx_ref[pl.ds(sgi, 1), :] = jnp.maximum(
                mx_ref[pl.ds(sgi, 1), :], jnp.max(hmask, axis=0, keepdims=True))
            sm_ref[pl.ds(sgi, 1), :] = (
                sm_ref[pl.ds(sgi, 1), :] + jnp.sum(hzero, axis=0, keepdims=True))
            cnt_ref[pl.ds(sgi, 1), :] = (
                cnt_ref[pl.ds(sgi, 1), :] + jnp.sum(m.astype(jnp.float32)))
            return carry

        lax.fori_loop(s_lo, s_hi + 1, seg, 0)

    return pl.pallas_call(
        body,
        grid=grid,
        in_specs=[
            pl.BlockSpec((2, _BLK, H), lambda i: (0, i, 0)),
            pl.BlockSpec((_BLK, H), lambda i: (i, 0)),
            pl.BlockSpec((H, H), lambda i: (0, 0)),
            pl.BlockSpec((1, H), lambda i: (0, 0)),
            pl.BlockSpec((H, H), lambda i: (0, 0)),
            pl.BlockSpec((_BLK, 1), lambda i: (i, 0)),
        ],
        out_specs=[
            pl.BlockSpec((_BLK, H), lambda i: (i, 0)),
            pl.BlockSpec((B, H), lambda i: (0, 0)),
            pl.BlockSpec((B, H), lambda i: (0, 0)),
            pl.BlockSpec((B, H), lambda i: (0, 0)),
        ],
        out_shape=[
            jax.ShapeDtypeStruct((N, H), jnp.float32),
            jax.ShapeDtypeStruct((B, H), jnp.float32),
            jax.ShapeDtypeStruct((B, H), jnp.float32),
            jax.ShapeDtypeStruct((B, H), jnp.float32),
        ],
    )(p.reshape(_NC, _NPAD, H), hprev, Wrel, brel_r, Wroot, batch_col)


def _mlp(mx1, sm1, cnt1, mx2, sm2, cnt2, mx3, sm3, cnt3,
         W1, b1_r, W2, b2_r, W3, b3_r):
    def body(mx1_ref, sm1_ref, cnt1_ref, mx2_ref, sm2_ref, cnt2_ref,
             mx3_ref, sm3_ref, cnt3_ref, w1_ref, b1_ref, w2_ref, b2_ref,
             w3_ref, b3_ref, out_ref, enc_ref):
        def gpart(mx_ref, sm_ref, cnt_ref):
            cnt = jnp.maximum(cnt_ref[...], 1.0)
            return jnp.concatenate([mx_ref[...], sm_ref[...] / cnt], axis=1)

        g = (gpart(mx1_ref, sm1_ref, cnt1_ref)
             + gpart(mx2_ref, sm2_ref, cnt2_ref)
             + gpart(mx3_ref, sm3_ref, cnt3_ref))
        enc_ref[...] = g
        z = lax.dot_general(g, w1_ref[...], (((1,), (1,)), ((), ())),
                            preferred_element_type=jnp.float32) + b1_ref[...]
        z = jnp.maximum(z, 0.0)
        z = lax.dot_general(z, w2_ref[...], (((1,), (1,)), ((), ())),
                            preferred_element_type=jnp.float32) + b2_ref[...]
        z = jnp.maximum(z, 0.0)
        z = jnp.sum(z * w3_ref[...], axis=1, keepdims=True) + b3_ref[0, 0]
        out_ref[...] = z

    return pl.pallas_call(
        body,
        out_shape=[
            jax.ShapeDtypeStruct((B, 1), jnp.float32),
            jax.ShapeDtypeStruct((B, 2 * H), jnp.float32),
        ],
    )(mx1, sm1, cnt1, mx2, sm2, cnt2, mx3, sm3, cnt3,
      W1, b1_r, W2, b2_r, W3, b3_r)


def kernel(x, edge_index, batch, Wrel1, brel1, Wroot1, Wrel2, brel2, Wroot2,
           Wrel3, brel3, Wroot3, W_lin1, b_lin1, W_lin2, b_lin2, W_lin3, b_lin3):
    pad = _EPAD - E
    dpad = (_CPT - _ITERS) * _CH
    src = jnp.concatenate([edge_index[0], jnp.zeros((pad,), jnp.int32)])
    src = jnp.concatenate(
        [src.reshape(_NT, _EPT), jnp.zeros((_NT, dpad), jnp.int32)], axis=1
    ).reshape(-1)
    dst = jnp.concatenate([edge_index[1], jnp.full((pad,), _DUMMY, jnp.int32)])
    dst = jnp.concatenate(
        [dst.reshape(_NT, _EPT), jnp.full((_NT, dpad), _DUMMY, jnp.int32)],
        axis=1).reshape(-1)
    zeros = jnp.zeros((_NPAD, H), jnp.float32)
    batch_col = batch.astype(jnp.float32).reshape(N, 1)

    p1 = _segsum_sc(x, src, dst, zeros)
    h1, mx1, sm1, cnt1 = _dense_pool(p1, x, Wrel1, brel1.reshape(1, H),
                                     Wroot1, batch_col)
    p2 = _segsum_sc(h1, src, dst, zeros)
    h2, mx2, sm2, cnt2 = _dense_pool(p2, h1, Wrel2, brel2.reshape(1, H),
                                     Wroot2, batch_col)
    p3 = _segsum_sc(h2, src, dst, zeros)
    h3, mx3, sm3, cnt3 = _dense_pool(p3, h2, Wrel3, brel3.reshape(1, H),
                                     Wroot3, batch_col)

    out, enc = _mlp(mx1, sm1, cnt1, mx2, sm2, cnt2, mx3, sm3, cnt3,
                    W_lin1, b_lin1.reshape(1, -1),
                    W_lin2, b_lin2.reshape(1, -1),
                    W_lin3, b_lin3.reshape(1, -1))
    return (out, lax.stop_gradient(enc))


# speedup vs baseline: 1.6853x; 1.2379x over previous
import functools

import jax
import jax.numpy as jnp
from jax import lax
from jax.experimental import pallas as pl
from jax.experimental.pallas import tpu as pltpu
from jax.experimental.pallas import tpu_sc as plsc

N = 10000
E = 320000
H = 128
B = 64

_NC = 2
_NS = 16
_NT = _NC * _NS
_CH = 80
_ITERS = 125
_CPT = _ITERS
_EPT = _ITERS * _CH
_EPAD = _NT * _EPT
_NPAD = 10240
_DUMMY = N + 100
_RPT = _NPAD // _NS


def _segsum_sc(h, idx2, zeros):
    mesh = plsc.VectorSubcoreMesh(core_axis_name="c", subcore_axis_name="s",
                                  num_cores=_NC, num_subcores=_NS)

    @functools.partial(
        pl.kernel,
        mesh=mesh,
        out_type=jax.ShapeDtypeStruct((_NC * _NPAD, H), jnp.float32),
        scratch_types=[
            pltpu.VMEM((2, _CH), jnp.int32),
            pltpu.VMEM((_CH, H), jnp.float32),
            pltpu.VMEM_SHARED((_NPAD, H), jnp.float32),
            pltpu.SemaphoreType.DMA,
            pltpu.SemaphoreType.DMA,
        ],
    )
    def k(h_hbm, idx_hbm, z_hbm, out_hbm, idxb, rows, acc, isem, gsem):
        c = lax.axis_index("c")
        s = lax.axis_index("s")
        w = c * _NS + s
        pltpu.sync_copy(z_hbm.at[pl.ds(s * _RPT, _RPT)],
                        acc.at[pl.ds(s * _RPT, _RPT)])
        base = w * _CPT
        plsc.subcore_barrier()

        def body(j, carry):
            pltpu.async_copy(idx_hbm.at[base + j], idxb, isem).wait()
            pltpu.async_copy(h_hbm.at[idxb.at[0]], rows, gsem).wait()
            pltpu.sync_copy(rows, acc.at[idxb.at[1]], add=True)
            return carry

        lax.fori_loop(0, _ITERS, body, 0)
        plsc.subcore_barrier()
        pltpu.sync_copy(acc.at[pl.ds(s * _RPT, _RPT)],
                        out_hbm.at[pl.ds(c * _NPAD + s * _RPT, _RPT)])

    return k(h, idx2, zeros)


_BLK = 1000


def _dense_pool(p, hprev, Wrel, brel_r, Wroot, batch_col):
    grid = (N // _BLK,)

    def body(p_ref, hp_ref, wr_ref, br_ref, wq_ref, b_ref,
             h_ref, mx_ref, sm_ref, cnt_ref):
        i = pl.program_id(0)
        agg = p_ref[0] + p_ref[1]
        hnew = lax.dot_general(agg, wr_ref[...], (((1,), (1,)), ((), ())),
                               preferred_element_type=jnp.float32)
        hnew = hnew + br_ref[...]
        hnew = hnew + lax.dot_general(hp_ref[...], wq_ref[...],
                                      (((1,), (1,)), ((), ())),
                                      preferred_element_type=jnp.float32)
        hnew = jnp.maximum(hnew, 0.0)
        h_ref[...] = hnew

        @pl.when(i == 0)
        def _init():
            mx_ref[...] = jnp.full((B, H), -jnp.inf, jnp.float32)
            sm_ref[...] = jnp.zeros((B, H), jnp.float32)
            cnt_ref[...] = jnp.zeros((B, H), jnp.float32)

        bb = b_ref[...]
        s_lo = b_ref[0, 0].astype(jnp.int32)
        s_hi = b_ref[_BLK - 1, 0].astype(jnp.int32)

        def seg(sgi, carry):
            m = bb == sgi.astype(jnp.float32)
            hmask = jnp.where(m, hnew, -jnp.inf)
            hzero = jnp.where(m, hnew, 0.0)
            mx_ref[pl.ds(sgi, 1), :] = jnp.maximum(
                mx_ref[pl.ds(sgi, 1), :], jnp.max(hmask, axis=0, keepdims=True))
            sm_ref[pl.ds(sgi, 1), :] = (
                sm_ref[pl.ds(sgi, 1), :] + jnp.sum(hzero, axis=0, keepdims=True))
            cnt_ref[pl.ds(sgi, 1), :] = (
                cnt_ref[pl.ds(sgi, 1), :] + jnp.sum(m.astype(jnp.float32)))
            return carry

        lax.fori_loop(s_lo, s_hi + 1, seg, 0)

    return pl.pallas_call(
        body,
        grid=grid,
        in_specs=[
            pl.BlockSpec((2, _BLK, H), lambda i: (0, i, 0)),
            pl.BlockSpec((_BLK, H), lambda i: (i, 0)),
            pl.BlockSpec((H, H), lambda i: (0, 0)),
            pl.BlockSpec((1, H), lambda i: (0, 0)),
            pl.BlockSpec((H, H), lambda i: (0, 0)),
            pl.BlockSpec((_BLK, 1), lambda i: (i, 0)),
        ],
        out_specs=[
            pl.BlockSpec((_BLK, H), lambda i: (i, 0)),
            pl.BlockSpec((B, H), lambda i: (0, 0)),
            pl.BlockSpec((B, H), lambda i: (0, 0)),
            pl.BlockSpec((B, H), lambda i: (0, 0)),
        ],
        out_shape=[
            jax.ShapeDtypeStruct((N, H), jnp.float32),
            jax.ShapeDtypeStruct((B, H), jnp.float32),
            jax.ShapeDtypeStruct((B, H), jnp.float32),
            jax.ShapeDtypeStruct((B, H), jnp.float32),
        ],
    )(p.reshape(_NC, _NPAD, H), hprev, Wrel, brel_r, Wroot, batch_col)


def _mlp(mx1, sm1, cnt1, mx2, sm2, cnt2, mx3, sm3, cnt3,
         W1, b1_r, W2, b2_r, W3, b3_r):
    def body(mx1_ref, sm1_ref, cnt1_ref, mx2_ref, sm2_ref, cnt2_ref,
             mx3_ref, sm3_ref, cnt3_ref, w1_ref, b1_ref, w2_ref, b2_ref,
             w3_ref, b3_ref, out_ref, enc_ref):
        def gpart(mx_ref, sm_ref, cnt_ref):
            cnt = jnp.maximum(cnt_ref[...], 1.0)
            return jnp.concatenate([mx_ref[...], sm_ref[...] / cnt], axis=1)

        g = (gpart(mx1_ref, sm1_ref, cnt1_ref)
             + gpart(mx2_ref, sm2_ref, cnt2_ref)
             + gpart(mx3_ref, sm3_ref, cnt3_ref))
        enc_ref[...] = g
        z = lax.dot_general(g, w1_ref[...], (((1,), (1,)), ((), ())),
                            preferred_element_type=jnp.float32) + b1_ref[...]
        z = jnp.maximum(z, 0.0)
        z = lax.dot_general(z, w2_ref[...], (((1,), (1,)), ((), ())),
                            preferred_element_type=jnp.float32) + b2_ref[...]
        z = jnp.maximum(z, 0.0)
        z = jnp.sum(z * w3_ref[...], axis=1, keepdims=True) + b3_ref[0, 0]
        out_ref[...] = z

    return pl.pallas_call(
        body,
        out_shape=[
            jax.ShapeDtypeStruct((B, 1), jnp.float32),
            jax.ShapeDtypeStruct((B, 2 * H), jnp.float32),
        ],
    )(mx1, sm1, cnt1, mx2, sm2, cnt2, mx3, sm3, cnt3,
      W1, b1_r, W2, b2_r, W3, b3_r)


def kernel(x, edge_index, batch, Wrel1, brel1, Wroot1, Wrel2, brel2, Wroot2,
           Wrel3, brel3, Wroot3, W_lin1, b_lin1, W_lin2, b_lin2, W_lin3, b_lin3):
    pad = _EPAD - E
    src = jnp.concatenate([edge_index[0], jnp.zeros((pad,), jnp.int32)])
    dst = jnp.concatenate([edge_index[1], jnp.full((pad,), _DUMMY, jnp.int32)])
    idx2 = jnp.stack(
        [src.reshape(_NT * _CPT, _CH), dst.reshape(_NT * _CPT, _CH)], axis=1)
    zeros = jnp.zeros((_NPAD, H), jnp.float32)
    batch_col = batch.astype(jnp.float32).reshape(N, 1)

    p1 = _segsum_sc(x, idx2, zeros)
    h1, mx1, sm1, cnt1 = _dense_pool(p1, x, Wrel1, brel1.reshape(1, H),
                                     Wroot1, batch_col)
    p2 = _segsum_sc(h1, idx2, zeros)
    h2, mx2, sm2, cnt2 = _dense_pool(p2, h1, Wrel2, brel2.reshape(1, H),
                                     Wroot2, batch_col)
    p3 = _segsum_sc(h2, idx2, zeros)
    h3, mx3, sm3, cnt3 = _dense_pool(p3, h2, Wrel3, brel3.reshape(1, H),
                                     Wroot3, batch_col)

    out, enc = _mlp(mx1, sm1, cnt1, mx2, sm2, cnt2, mx3, sm3, cnt3,
                    W_lin1, b_lin1.reshape(1, -1),
                    W_lin2, b_lin2.reshape(1, -1),
                    W_lin3, b_lin3.reshape(1, -1))
    return (out, lax.stop_gradient(enc))


# speedup vs baseline: 1.7331x; 1.0284x over previous
import functools

import jax
import jax.numpy as jnp
from jax import lax
from jax.experimental import pallas as pl
from jax.experimental.pallas import tpu as pltpu
from jax.experimental.pallas import tpu_sc as plsc

N = 10000
E = 320000
H = 128
B = 64

_NC = 2
_NS = 16
_NT = _NC * _NS
_CH = 80
_ITERS = 125
_CPT = _ITERS
_EPT = _ITERS * _CH
_EPAD = _NT * _EPT
_NPAD = 10240
_DUMMY = N + 100
_RPT = _NPAD // _NS


def _segsum_sc(h, src, dst, zeros):
    mesh = plsc.VectorSubcoreMesh(core_axis_name="c", subcore_axis_name="s",
                                  num_cores=_NC, num_subcores=_NS)

    @functools.partial(
        pl.kernel,
        mesh=mesh,
        out_type=jax.ShapeDtypeStruct((_NC * _NPAD, H), jnp.float32),
        scratch_types=[
            pltpu.VMEM((_CH,), jnp.int32),
            pltpu.VMEM((_CH,), jnp.int32),
            pltpu.VMEM((_CH, H), jnp.float32),
            pltpu.VMEM_SHARED((_NPAD, H), jnp.float32),
            pltpu.SemaphoreType.DMA,
            pltpu.SemaphoreType.DMA,
        ],
    )
    def k(h_hbm, src_hbm, dst_hbm, z_hbm, out_hbm, sidx, didx, rows, acc,
          isem, gsem):
        c = lax.axis_index("c")
        s = lax.axis_index("s")
        w = c * _NS + s
        pltpu.sync_copy(z_hbm.at[pl.ds(s * _RPT, _RPT)],
                        acc.at[pl.ds(s * _RPT, _RPT)])
        base = w * _CPT * _CH
        plsc.subcore_barrier()

        def idx_pair(j):
            return (pltpu.make_async_copy(
                        src_hbm.at[pl.ds(base + j * _CH, _CH)], sidx, isem),
                    pltpu.make_async_copy(
                        dst_hbm.at[pl.ds(base + j * _CH, _CH)], didx, isem))

        def body(j, carry):
            for d in idx_pair(j):
                d.start()
            for d in idx_pair(j):
                d.wait()
            pltpu.async_copy(h_hbm.at[sidx], rows, gsem).wait()
            pltpu.sync_copy(rows, acc.at[didx], add=True)
            return carry

        lax.fori_loop(0, _ITERS, body, 0)
        plsc.subcore_barrier()
        pltpu.sync_copy(acc.at[pl.ds(s * _RPT, _RPT)],
                        out_hbm.at[pl.ds(c * _NPAD + s * _RPT, _RPT)])

    return k(h, src, dst, zeros)


_BLK = 1000


def _dense_pool(p, hprev, Wrel, brel_r, Wroot, batch_col):
    grid = (N // _BLK,)

    def body(p_ref, hp_ref, wr_ref, br_ref, wq_ref, b_ref,
             h_ref, mx_ref, sm_ref, cnt_ref):
        i = pl.program_id(0)
        agg = p_ref[0] + p_ref[1]
        hnew = lax.dot_general(agg, wr_ref[...], (((1,), (1,)), ((), ())),
                               preferred_element_type=jnp.float32)
        hnew = hnew + br_ref[...]
        hnew = hnew + lax.dot_general(hp_ref[...], wq_ref[...],
                                      (((1,), (1,)), ((), ())),
                                      preferred_element_type=jnp.float32)
        hnew = jnp.maximum(hnew, 0.0)
        h_ref[...] = hnew

        @pl.when(i == 0)
        def _init():
            mx_ref[...] = jnp.full((B, H), -jnp.inf, jnp.float32)
            sm_ref[...] = jnp.zeros((B, H), jnp.float32)
            cnt_ref[...] = jnp.zeros((B, H), jnp.float32)

        bb = b_ref[...]
        s_lo = b_ref[0, 0].astype(jnp.int32)
        s_hi = b_ref[_BLK - 1, 0].astype(jnp.int32)

        def seg(sgi, carry):
            m = bb == sgi.astype(jnp.float32)
            hmask = jnp.where(m, hnew, -jnp.inf)
            hzero = jnp.where(m, hnew, 0.0)
            mx_ref[pl.ds(sgi, 1), :] = jnp.maximum(
                mx_ref[pl.ds(sgi, 1), :], jnp.max(hmask, axis=0, keepdims=True))
            sm_ref[pl.ds(sgi, 1), :] = (
                sm_ref[pl.ds(sgi, 1), :] + jnp.sum(hzero, axis=0, keepdims=True))
            cnt_ref[pl.ds(sgi, 1), :] = (
                cnt_ref[pl.ds(sgi, 1), :] + jnp.sum(m.astype(jnp.float32)))
            return carry

        lax.fori_loop(s_lo, s_hi + 1, seg, 0)

    return pl.pallas_call(
        body,
        grid=grid,
        in_specs=[
            pl.BlockSpec((2, _BLK, H), lambda i: (0, i, 0)),
            pl.BlockSpec((_BLK, H), lambda i: (i, 0)),
            pl.BlockSpec((H, H), lambda i: (0, 0)),
            pl.BlockSpec((1, H), lambda i: (0, 0)),
            pl.BlockSpec((H, H), lambda i: (0, 0)),
            pl.BlockSpec((_BLK, 1), lambda i: (i, 0)),
        ],
        out_specs=[
            pl.BlockSpec((_BLK, H), lambda i: (i, 0)),
            pl.BlockSpec((B, H), lambda i: (0, 0)),
            pl.BlockSpec((B, H), lambda i: (0, 0)),
            pl.BlockSpec((B, H), lambda i: (0, 0)),
        ],
        out_shape=[
            jax.ShapeDtypeStruct((N, H), jnp.float32),
            jax.ShapeDtypeStruct((B, H), jnp.float32),
            jax.ShapeDtypeStruct((B, H), jnp.float32),
            jax.ShapeDtypeStruct((B, H), jnp.float32),
        ],
    )(p.reshape(_NC, _NPAD, H), hprev, Wrel, brel_r, Wroot, batch_col)


def _mlp(mx1, sm1, cnt1, mx2, sm2, cnt2, mx3, sm3, cnt3,
         W1, b1_r, W2, b2_r, W3, b3_r):
    def body(mx1_ref, sm1_ref, cnt1_ref, mx2_ref, sm2_ref, cnt2_ref,
             mx3_ref, sm3_ref, cnt3_ref, w1_ref, b1_ref, w2_ref, b2_ref,
             w3_ref, b3_ref, out_ref, enc_ref):
        def gpart(mx_ref, sm_ref, cnt_ref):
            cnt = jnp.maximum(cnt_ref[...], 1.0)
            return jnp.concatenate([mx_ref[...], sm_ref[...] / cnt], axis=1)

        g = (gpart(mx1_ref, sm1_ref, cnt1_ref)
             + gpart(mx2_ref, sm2_ref, cnt2_ref)
             + gpart(mx3_ref, sm3_ref, cnt3_ref))
        enc_ref[...] = g
        z = lax.dot_general(g, w1_ref[...], (((1,), (1,)), ((), ())),
                            preferred_element_type=jnp.float32) + b1_ref[...]
        z = jnp.maximum(z, 0.0)
        z = lax.dot_general(z, w2_ref[...], (((1,), (1,)), ((), ())),
                            preferred_element_type=jnp.float32) + b2_ref[...]
        z = jnp.maximum(z, 0.0)
        z = jnp.sum(z * w3_ref[...], axis=1, keepdims=True) + b3_ref[0, 0]
        out_ref[...] = z

    return pl.pallas_call(
        body,
        out_shape=[
            jax.ShapeDtypeStruct((B, 1), jnp.float32),
            jax.ShapeDtypeStruct((B, 2 * H), jnp.float32),
        ],
    )(mx1, sm1, cnt1, mx2, sm2, cnt2, mx3, sm3, cnt3,
      W1, b1_r, W2, b2_r, W3, b3_r)


def kernel(x, edge_index, batch, Wrel1, brel1, Wroot1, Wrel2, brel2, Wroot2,
           Wrel3, brel3, Wroot3, W_lin1, b_lin1, W_lin2, b_lin2, W_lin3, b_lin3):
    pad = _EPAD - E
    src = jnp.concatenate([edge_index[0], jnp.zeros((pad,), jnp.int32)])
    dst = jnp.concatenate([edge_index[1], jnp.full((pad,), _DUMMY, jnp.int32)])
    zeros = jnp.zeros((_NPAD, H), jnp.float32)
    batch_col = batch.astype(jnp.float32).reshape(N, 1)

    p1 = _segsum_sc(x, src, dst, zeros)
    h1, mx1, sm1, cnt1 = _dense_pool(p1, x, Wrel1, brel1.reshape(1, H),
                                     Wroot1, batch_col)
    p2 = _segsum_sc(h1, src, dst, zeros)
    h2, mx2, sm2, cnt2 = _dense_pool(p2, h1, Wrel2, brel2.reshape(1, H),
                                     Wroot2, batch_col)
    p3 = _segsum_sc(h2, src, dst, zeros)
    h3, mx3, sm3, cnt3 = _dense_pool(p3, h2, Wrel3, brel3.reshape(1, H),
                                     Wroot3, batch_col)

    out, enc = _mlp(mx1, sm1, cnt1, mx2, sm2, cnt2, mx3, sm3, cnt3,
                    W_lin1, b_lin1.reshape(1, -1),
                    W_lin2, b_lin2.reshape(1, -1),
                    W_lin3, b_lin3.reshape(1, -1))
    return (out, lax.stop_gradient(enc))
